# Initial kernel scaffold; baseline (speedup 1.0000x reference)
import jax
import jax.numpy as jnp
from jax.experimental import pallas as pl


def kernel(inputs, gate_w, expert_w):
    raise NotImplementedError("write your pallas kernel here")


# speedup vs baseline: 2.9657x; 2.9657x over previous
import jax
import jax.numpy as jnp
from jax.experimental import pallas as pl
from jax.experimental.pallas import tpu as pltpu

_TOP_K = 2
_TM = 512


def _moe_dense_body(x_ref, wgt_ref, w_ref, o_ref):
    x = x_ref[...]
    num_experts = w_ref.shape[0]
    acc = None
    for e in range(num_experts):
        y = jnp.dot(x, w_ref[e], preferred_element_type=jnp.float32)
        term = wgt_ref[:, e][:, None] * y
        acc = term if acc is None else acc + term
    o_ref[...] = acc


def kernel(inputs, gate_w, expert_w):
    B, T, C = inputs.shape
    E = gate_w.shape[0]
    N = B * T
    x = inputs.reshape(N, C)

    gate_logits = x @ gate_w.T
    topk_vals, topk_idx = jax.lax.top_k(gate_logits, _TOP_K)
    weights = jax.nn.softmax(topk_vals.astype(jnp.float32), axis=1)

    wgt = jnp.zeros((N, E), jnp.float32).at[
        jnp.arange(N, dtype=jnp.int32)[:, None], topk_idx].set(weights)

    x_bf = x.astype(jnp.bfloat16)
    w_bf = jnp.swapaxes(expert_w, 1, 2).astype(jnp.bfloat16)

    tm = _TM if N % _TM == 0 else N
    out = pl.pallas_call(
        _moe_dense_body,
        out_shape=jax.ShapeDtypeStruct((N, C), jnp.float32),
        grid=(N // tm,),
        in_specs=[
            pl.BlockSpec((tm, C), lambda t: (t, 0)),
            pl.BlockSpec((tm, E), lambda t: (t, 0)),
            pl.BlockSpec((E, C, C), lambda t: (0, 0, 0)),
        ],
        out_specs=pl.BlockSpec((tm, C), lambda t: (t, 0)),
        compiler_params=pltpu.CompilerParams(
            dimension_semantics=("parallel",),
            vmem_limit_bytes=60 * 1024 * 1024,
        ),
    )(x_bf, wgt, w_bf)

    return out.astype(inputs.dtype).reshape(B, T, C)


# speedup vs baseline: 4.3826x; 1.4778x over previous
import jax
import jax.numpy as jnp
from jax.experimental import pallas as pl
from jax.experimental.pallas import tpu as pltpu

_TOP_K = 2
_TM = 512


def _moe_dense_body(x_ref, wgt_ref, w_ref, o_ref):
    x = x_ref[...].astype(jnp.bfloat16)
    num_experts = w_ref.shape[0]
    acc = None
    for e in range(num_experts):
        y = jax.lax.dot_general(
            x, w_ref[e], (((1,), (1,)), ((), ())),
            preferred_element_type=jnp.float32)
        term = wgt_ref[:, e][:, None] * y
        acc = term if acc is None else acc + term
    o_ref[...] = acc


def kernel(inputs, gate_w, expert_w):
    B, T, C = inputs.shape
    E = gate_w.shape[0]
    N = B * T
    x = inputs.reshape(N, C)

    gate_logits = x @ gate_w.T
    topk_vals, topk_idx = jax.lax.top_k(gate_logits, _TOP_K)
    weights = jax.nn.softmax(topk_vals.astype(jnp.float32), axis=1)

    wgt = jnp.sum(
        (topk_idx[:, :, None] == jnp.arange(E, dtype=topk_idx.dtype))
        * weights[:, :, None],
        axis=1)

    w_bf = expert_w.astype(jnp.bfloat16)

    tm = _TM if N % _TM == 0 else N
    out = pl.pallas_call(
        _moe_dense_body,
        out_shape=jax.ShapeDtypeStruct((N, C), jnp.float32),
        grid=(N // tm,),
        in_specs=[
            pl.BlockSpec((tm, C), lambda t: (t, 0)),
            pl.BlockSpec((tm, E), lambda t: (t, 0)),
            pl.BlockSpec((E, C, C), lambda t: (0, 0, 0)),
        ],
        out_specs=pl.BlockSpec((tm, C), lambda t: (t, 0)),
        compiler_params=pltpu.CompilerParams(
            dimension_semantics=("parallel",),
            vmem_limit_bytes=60 * 1024 * 1024,
        ),
    )(x, wgt, w_bf)

    return out.astype(inputs.dtype).reshape(B, T, C)
